# baseline (speedup 1.0000x reference)
import jax
import jax.numpy as jnp
from jax import lax
from jax.experimental import pallas as pl
from jax.experimental.pallas import tpu as pltpu

_N = 96
_BX = 8
_G = _N // _BX


def _slab_copy(u_hbm, u_vmem, sems, j):
    return pltpu.make_async_copy(
        u_hbm.at[pl.ds(j * _BX, _BX)],
        u_vmem.at[pl.ds(j * _BX, _BX)],
        sems.at[j],
    )


def _body(u_hbm, t_ref, f_ref, out_ref, u_vmem, sems):
    i = pl.program_id(0)

    @pl.when(i == 0)
    def _():
        _slab_copy(u_hbm, u_vmem, sems, _G - 1).start()
        _slab_copy(u_hbm, u_vmem, sems, 0).start()
        _slab_copy(u_hbm, u_vmem, sems, 1).start()
        _slab_copy(u_hbm, u_vmem, sems, 2).start()
        _slab_copy(u_hbm, u_vmem, sems, _G - 1).wait()
        _slab_copy(u_hbm, u_vmem, sems, 0).wait()
        _slab_copy(u_hbm, u_vmem, sems, 1).wait()

    @pl.when((i > 0) & (i < _G - 3))
    def _():
        pltpu.make_async_copy(
            u_hbm.at[pl.ds((i + 2) * _BX, _BX)],
            u_vmem.at[pl.ds((i + 2) * _BX, _BX)],
            sems.at[i + 2],
        ).start()

    @pl.when((i > 0) & (i < _G - 2))
    def _():
        pltpu.make_async_copy(
            u_hbm.at[pl.ds((i + 1) * _BX, _BX)],
            u_vmem.at[pl.ds((i + 1) * _BX, _BX)],
            sems.at[i + 1],
        ).wait()

    neg_a = f_ref[1, 0]
    d_plus_a = f_ref[1, 13] - f_ref[1, 0]

    x0 = i * _BX
    lo = (x0 + (_N - 1)) % _N
    hi = (x0 + _BX) % _N
    t = t_ref[...]

    Ex = jnp.concatenate(
        [
            u_vmem[pl.ds(lo, 1)] + u_vmem[pl.ds(x0, 1)],
            u_vmem[pl.ds(x0, _BX - 1)] + u_vmem[pl.ds(x0 + 1, _BX - 1)],
            u_vmem[pl.ds(x0 + _BX - 1, 1)] + u_vmem[pl.ds(hi, 1)],
            u_vmem[pl.ds(hi, 1)] + u_vmem[pl.ds(hi + 1, 1)],
        ],
        axis=0,
    )
    Exy = Ex + jnp.roll(Ex, -1, 1)
    E = Exy + jnp.roll(Exy, -1, 2)

    e_yz = {
        (1, 1): E,
        (1, 0): jnp.roll(E, 1, 2),
        (0, 1): jnp.roll(E, 1, 1),
    }
    e_yz[(0, 0)] = jnp.roll(e_yz[(1, 0)], 1, 1)

    acc_i = jnp.zeros((_BX, _N, _N), jnp.int32)
    acc = jnp.zeros((_BX, _N, _N), jnp.float32)
    for p1 in (0, 1):
        for p2 in (0, 1):
            eyz = e_yz[(p1, p2)]
            eyz_x0 = lax.bitcast_convert_type(eyz[1 : _BX + 1], jnp.int32)
            eyz_xm1 = lax.bitcast_convert_type(eyz[:_BX], jnp.int32)
            for p0 in (0, 1):
                e = p0 * 4 + p1 * 2 + p2
                m = (t << (31 - e)) >> 31
                acc_i = acc_i + m
                acc = acc + lax.bitcast_convert_type(
                    m & (eyz_x0 if p0 else eyz_xm1), jnp.float32
                )
    U0 = u_vmem[pl.ds(x0, _BX)]
    pc = (-acc_i).astype(jnp.float32)
    out_ref[...] = d_plus_a * (U0 * pc) + neg_a * acc


def kernel(U, H8types, filters):
    return pl.pallas_call(
        _body,
        grid=(_G,),
        in_specs=[
            pl.BlockSpec(memory_space=pltpu.MemorySpace.HBM),
            pl.BlockSpec((_BX, _N, _N), lambda i: (i, 0, 0)),
            pl.BlockSpec((256, 27), lambda i: (0, 0)),
        ],
        out_specs=pl.BlockSpec((_BX, _N, _N), lambda i: (i, 0, 0)),
        out_shape=jax.ShapeDtypeStruct(U.shape, U.dtype),
        scratch_shapes=[
            pltpu.MemorySpace.VMEM((_N, _N, _N), jnp.float32),
            pltpu.SemaphoreType.DMA((_G,)),
        ],
    )(U, H8types, filters)


# speedup vs baseline: 1.1329x; 1.1329x over previous
import jax
import jax.numpy as jnp
from jax import lax
from jax.experimental import pallas as pl
from jax.experimental.pallas import tpu as pltpu

_N = 96
_BX = 16
_G = _N // _BX


def _slab_copy(u_hbm, u_vmem, sems, j):
    return pltpu.make_async_copy(
        u_hbm.at[pl.ds(j * _BX, _BX)],
        u_vmem.at[pl.ds(j * _BX, _BX)],
        sems.at[j],
    )


def _body(u_hbm, t_ref, f_ref, out_ref, u_vmem, sems):
    i = pl.program_id(0)

    @pl.when(i == 0)
    def _():
        _slab_copy(u_hbm, u_vmem, sems, _G - 1).start()
        _slab_copy(u_hbm, u_vmem, sems, 0).start()
        _slab_copy(u_hbm, u_vmem, sems, 1).start()
        _slab_copy(u_hbm, u_vmem, sems, 2).start()
        _slab_copy(u_hbm, u_vmem, sems, _G - 1).wait()
        _slab_copy(u_hbm, u_vmem, sems, 0).wait()
        _slab_copy(u_hbm, u_vmem, sems, 1).wait()

    @pl.when((i > 0) & (i < _G - 3))
    def _():
        pltpu.make_async_copy(
            u_hbm.at[pl.ds((i + 2) * _BX, _BX)],
            u_vmem.at[pl.ds((i + 2) * _BX, _BX)],
            sems.at[i + 2],
        ).start()

    @pl.when((i > 0) & (i < _G - 2))
    def _():
        pltpu.make_async_copy(
            u_hbm.at[pl.ds((i + 1) * _BX, _BX)],
            u_vmem.at[pl.ds((i + 1) * _BX, _BX)],
            sems.at[i + 1],
        ).wait()

    neg_a = f_ref[1, 0]
    d_plus_a = f_ref[1, 13] - f_ref[1, 0]

    x0 = i * _BX
    lo = (x0 + (_N - 1)) % _N
    hi = (x0 + _BX) % _N
    t = t_ref[...]

    Ex = jnp.concatenate(
        [
            u_vmem[pl.ds(lo, 1)] + u_vmem[pl.ds(x0, 1)],
            u_vmem[pl.ds(x0, _BX - 1)] + u_vmem[pl.ds(x0 + 1, _BX - 1)],
            u_vmem[pl.ds(x0 + _BX - 1, 1)] + u_vmem[pl.ds(hi, 1)],
            u_vmem[pl.ds(hi, 1)] + u_vmem[pl.ds(hi + 1, 1)],
        ],
        axis=0,
    )
    Exy = Ex + jnp.roll(Ex, -1, 1)
    E = Exy + jnp.roll(Exy, -1, 2)

    e_yz = {
        (1, 1): E,
        (1, 0): jnp.roll(E, 1, 2),
        (0, 1): jnp.roll(E, 1, 1),
    }
    e_yz[(0, 0)] = jnp.roll(e_yz[(1, 0)], 1, 1)

    acc_i = jnp.zeros((_BX, _N, _N), jnp.int32)
    acc = jnp.zeros((_BX, _N, _N), jnp.float32)
    for p1 in (0, 1):
        for p2 in (0, 1):
            eyz = e_yz[(p1, p2)]
            eyz_x0 = lax.bitcast_convert_type(eyz[1 : _BX + 1], jnp.int32)
            eyz_xm1 = lax.bitcast_convert_type(eyz[:_BX], jnp.int32)
            for p0 in (0, 1):
                e = p0 * 4 + p1 * 2 + p2
                m = (t << (31 - e)) >> 31
                acc_i = acc_i + m
                acc = acc + lax.bitcast_convert_type(
                    m & (eyz_x0 if p0 else eyz_xm1), jnp.float32
                )
    U0 = u_vmem[pl.ds(x0, _BX)]
    pc = (-acc_i).astype(jnp.float32)
    out_ref[...] = d_plus_a * (U0 * pc) + neg_a * acc


def kernel(U, H8types, filters):
    return pl.pallas_call(
        _body,
        grid=(_G,),
        in_specs=[
            pl.BlockSpec(memory_space=pltpu.MemorySpace.HBM),
            pl.BlockSpec((_BX, _N, _N), lambda i: (i, 0, 0)),
            pl.BlockSpec((256, 27), lambda i: (0, 0)),
        ],
        out_specs=pl.BlockSpec((_BX, _N, _N), lambda i: (i, 0, 0)),
        out_shape=jax.ShapeDtypeStruct(U.shape, U.dtype),
        scratch_shapes=[
            pltpu.MemorySpace.VMEM((_N, _N, _N), jnp.float32),
            pltpu.SemaphoreType.DMA((_G,)),
        ],
    )(U, H8types, filters)
